# Initial kernel scaffold; baseline (speedup 1.0000x reference)
import functools

import jax
import jax.numpy as jnp
from jax import lax
from jax.experimental import pallas as pl
from jax.experimental.pallas import tpu as pltpu
from jax.experimental.pallas import tpu_sc as plsc

N_ITEMS = 100000
AUDIO_DIM = 128
D_MODEL = 128
BATCH = 4096
HIST = 200


_BLK = 512


def _project_body(x_ref, w_ref, b_ref, o_ref):
    x = x_ref[...]
    w = w_ref[...]
    y = lax.dot_general(x, w, (((1,), (1,)), ((), ())),
                        preferred_element_type=jnp.float32)
    y = y + b_ref[...]
    ss = jnp.sum(y * y, axis=1, keepdims=True)
    o_ref[...] = y / jnp.maximum(jnp.sqrt(ss), 1e-12)


def _project_table(padded_table, W, b):
    n_rows = padded_table.shape[0]
    grid = pl.cdiv(n_rows, _BLK)
    return pl.pallas_call(
        _project_body,
        grid=(grid,),
        in_specs=[
            pl.BlockSpec((_BLK, AUDIO_DIM), lambda i: (i, 0)),
            pl.BlockSpec((D_MODEL, AUDIO_DIM), lambda i: (0, 0)),
            pl.BlockSpec((1, D_MODEL), lambda i: (0, 0)),
        ],
        out_specs=pl.BlockSpec((_BLK, D_MODEL), lambda i: (i, 0)),
        out_shape=jax.ShapeDtypeStruct((n_rows, D_MODEL), jnp.float32),
    )(padded_table, W, b.reshape(1, D_MODEL))


_INFO = plsc.get_sparse_core_info()
_NC, _NS = _INFO.num_cores, _INFO.num_subcores
_NW = _NC * _NS
_BL = BATCH * HIST
_G = 128
_CH = _BL // (_NW * _G)
_NBUF = 4


def _gather_body(ids_hbm, table_hbm, out_hbm, idx_v,
                 b0, b1, b2, b3, s0, s1, s2, s3):
    bufs = (b0, b1, b2, b3)
    sems = (s0, s1, s2, s3)
    wid = lax.axis_index("s") * _NC + lax.axis_index("c")
    base = wid * (_CH * _G)

    pltpu.sync_copy(ids_hbm.at[wid], idx_v)

    for b in range(_NBUF):
        pltpu.async_copy(table_hbm.at[idx_v.at[b]], bufs[b], sems[b])

    def outer(i, carry):
        j0 = i * _NBUF
        for b in range(_NBUF):
            j = j0 + b
            pltpu.make_async_copy(table_hbm.at[idx_v.at[j]],
                                  bufs[b], sems[b]).wait()
            pltpu.sync_copy(bufs[b], out_hbm.at[pl.ds(base + j * _G, _G)])
            pltpu.async_copy(table_hbm.at[idx_v.at[j + _NBUF]],
                             bufs[b], sems[b])
        return carry

    lax.fori_loop(0, (_CH - _NBUF) // _NBUF, outer, 0)

    j0 = _CH - _NBUF
    for b in range(_NBUF):
        j = j0 + b
        pltpu.make_async_copy(table_hbm.at[idx_v.at[j]],
                              bufs[b], sems[b]).wait()
        pltpu.sync_copy(bufs[b], out_hbm.at[pl.ds(base + j * _G, _G)])


@functools.partial(
    pl.kernel,
    out_type=jax.ShapeDtypeStruct((_BL, D_MODEL), jnp.float32),
    mesh=plsc.VectorSubcoreMesh(core_axis_name="c", subcore_axis_name="s"),
    scratch_types=[
        pltpu.VMEM((_CH, _G), jnp.int32),
        pltpu.VMEM((_G, D_MODEL), jnp.float32),
        pltpu.VMEM((_G, D_MODEL), jnp.float32),
        pltpu.VMEM((_G, D_MODEL), jnp.float32),
        pltpu.VMEM((_G, D_MODEL), jnp.float32),
        pltpu.SemaphoreType.DMA,
        pltpu.SemaphoreType.DMA,
        pltpu.SemaphoreType.DMA,
        pltpu.SemaphoreType.DMA,
    ],
)
def _gather_rows(ids_hbm, table_hbm, out_hbm, *scratch):
    _gather_body(ids_hbm, table_hbm, out_hbm, *scratch)


def kernel(dense_ids, padded_table, W, b):
    proj = _project_table(padded_table, W, b)
    ids = (dense_ids.astype(jnp.int32) + 1).reshape(_NW, _CH, _G)
    rows = _gather_rows(ids, proj)
    return rows.reshape(BATCH, HIST, D_MODEL)


# speedup vs baseline: 7.4302x; 7.4302x over previous
import functools

import jax
import jax.numpy as jnp
from jax import lax
from jax.experimental import pallas as pl
from jax.experimental.pallas import tpu as pltpu
from jax.experimental.pallas import tpu_sc as plsc

N_ITEMS = 100000
AUDIO_DIM = 128
D_MODEL = 128
BATCH = 4096
HIST = 200


_BLK = 512


def _project_body(x_ref, w_ref, b_ref, o_ref):
    x = x_ref[...]
    w = w_ref[...]
    y = lax.dot_general(x, w, (((1,), (1,)), ((), ())),
                        preferred_element_type=jnp.float32)
    y = y + b_ref[...]
    ss = jnp.sum(y * y, axis=1, keepdims=True)
    o_ref[...] = y / jnp.maximum(jnp.sqrt(ss), 1e-12)


def _project_table(padded_table, W, b):
    n_rows = padded_table.shape[0]
    grid = pl.cdiv(n_rows, _BLK)
    return pl.pallas_call(
        _project_body,
        grid=(grid,),
        in_specs=[
            pl.BlockSpec((_BLK, AUDIO_DIM), lambda i: (i, 0)),
            pl.BlockSpec((D_MODEL, AUDIO_DIM), lambda i: (0, 0)),
            pl.BlockSpec((1, D_MODEL), lambda i: (0, 0)),
        ],
        out_specs=pl.BlockSpec((_BLK, D_MODEL), lambda i: (i, 0)),
        out_shape=jax.ShapeDtypeStruct((n_rows, D_MODEL), jnp.float32),
    )(padded_table, W, b.reshape(1, D_MODEL))


_NC, _NS = 2, 16
_NW = _NC * _NS
_BL = BATCH * HIST
_G = 128
_CH = _BL // (_NW * _G)
_NBUF = 4


def _gather_body(ids_hbm, table_hbm, out_hbm, idx_v,
                 b0, b1, b2, b3, s0, s1, s2, s3):
    bufs = (b0, b1, b2, b3)
    sems = (s0, s1, s2, s3)
    wid = lax.axis_index("s") * _NC + lax.axis_index("c")
    base = wid * (_CH * _G)

    pltpu.sync_copy(ids_hbm.at[wid], idx_v)

    for b in range(_NBUF):
        pltpu.async_copy(table_hbm.at[idx_v.at[b]], bufs[b], sems[b])

    def outer(i, carry):
        j0 = i * _NBUF
        for b in range(_NBUF):
            j = j0 + b
            pltpu.make_async_copy(table_hbm.at[idx_v.at[j]],
                                  bufs[b], sems[b]).wait()
            pltpu.sync_copy(bufs[b], out_hbm.at[pl.ds(base + j * _G, _G)])
            pltpu.async_copy(table_hbm.at[idx_v.at[j + _NBUF]],
                             bufs[b], sems[b])
        return carry

    lax.fori_loop(0, (_CH - _NBUF) // _NBUF, outer, 0)

    j0 = _CH - _NBUF
    for b in range(_NBUF):
        j = j0 + b
        pltpu.make_async_copy(table_hbm.at[idx_v.at[j]],
                              bufs[b], sems[b]).wait()
        pltpu.sync_copy(bufs[b], out_hbm.at[pl.ds(base + j * _G, _G)])


@functools.cache
def _gather_rows_fn():
    @functools.partial(
        pl.kernel,
        out_type=jax.ShapeDtypeStruct((_BL, D_MODEL), jnp.float32),
        mesh=plsc.VectorSubcoreMesh(core_axis_name="c", subcore_axis_name="s"),
        scratch_types=[
            pltpu.VMEM((_CH, _G), jnp.int32),
            pltpu.VMEM((_G, D_MODEL), jnp.float32),
            pltpu.VMEM((_G, D_MODEL), jnp.float32),
            pltpu.VMEM((_G, D_MODEL), jnp.float32),
            pltpu.VMEM((_G, D_MODEL), jnp.float32),
            pltpu.SemaphoreType.DMA,
            pltpu.SemaphoreType.DMA,
            pltpu.SemaphoreType.DMA,
            pltpu.SemaphoreType.DMA,
        ],
    )
    def _gather_rows(ids_hbm, table_hbm, out_hbm, *scratch):
        _gather_body(ids_hbm, table_hbm, out_hbm, *scratch)

    return _gather_rows


def kernel(dense_ids, padded_table, W, b):
    proj = _project_table(padded_table, W, b)
    ids = (dense_ids.astype(jnp.int32) + 1).reshape(_NW, _CH, _G)
    rows = _gather_rows_fn()(ids, proj)
    return rows.reshape(BATCH, HIST, D_MODEL)


# speedup vs baseline: 9.0517x; 1.2182x over previous
import functools

import jax
import jax.numpy as jnp
from jax import lax
from jax.experimental import pallas as pl
from jax.experimental.pallas import tpu as pltpu
from jax.experimental.pallas import tpu_sc as plsc

N_ITEMS = 100000
AUDIO_DIM = 128
D_MODEL = 128
BATCH = 4096
HIST = 200


_BLK = 2048


def _project_body(x_ref, w_ref, b_ref, o_ref):
    x = x_ref[...]
    w = w_ref[...]
    y = lax.dot_general(x, w, (((1,), (1,)), ((), ())),
                        preferred_element_type=jnp.float32)
    y = y + b_ref[...]
    ss = jnp.sum(y * y, axis=1, keepdims=True)
    o_ref[...] = y * lax.rsqrt(jnp.maximum(ss, 1e-24))


def _project_table(padded_table, W, b):
    n_rows = padded_table.shape[0]
    grid = pl.cdiv(n_rows, _BLK)
    return pl.pallas_call(
        _project_body,
        grid=(grid,),
        in_specs=[
            pl.BlockSpec((_BLK, AUDIO_DIM), lambda i: (i, 0)),
            pl.BlockSpec((D_MODEL, AUDIO_DIM), lambda i: (0, 0)),
            pl.BlockSpec((1, D_MODEL), lambda i: (0, 0)),
        ],
        out_specs=pl.BlockSpec((_BLK, D_MODEL), lambda i: (i, 0)),
        out_shape=jax.ShapeDtypeStruct((n_rows, D_MODEL), jnp.float32),
    )(padded_table, W, b.reshape(1, D_MODEL))


_NC, _NS = 2, 16
_NW = _NC * _NS
_BL = BATCH * HIST
_G = 128
_CH = _BL // (_NW * _G)
_NBUF = 4


def _gather_body(ids_hbm, table_hbm, out_hbm, idx_v,
                 b0, b1, b2, b3, s0, s1, s2, s3):
    bufs = (b0, b1, b2, b3)
    sems = (s0, s1, s2, s3)
    wid = lax.axis_index("s") * _NC + lax.axis_index("c")
    base = wid * (_CH * _G)

    pltpu.sync_copy(ids_hbm.at[wid], idx_v)

    for b in range(_NBUF):
        pltpu.async_copy(table_hbm.at[idx_v.at[b]], bufs[b], sems[b])

    def outer(i, carry):
        j0 = i * _NBUF
        for b in range(_NBUF):
            j = j0 + b
            pltpu.make_async_copy(table_hbm.at[idx_v.at[j]],
                                  bufs[b], sems[b]).wait()
            pltpu.sync_copy(bufs[b], out_hbm.at[pl.ds(base + j * _G, _G)])
            pltpu.async_copy(table_hbm.at[idx_v.at[j + _NBUF]],
                             bufs[b], sems[b])
        return carry

    lax.fori_loop(0, (_CH - _NBUF) // _NBUF, outer, 0)

    j0 = _CH - _NBUF
    for b in range(_NBUF):
        j = j0 + b
        pltpu.make_async_copy(table_hbm.at[idx_v.at[j]],
                              bufs[b], sems[b]).wait()
        pltpu.sync_copy(bufs[b], out_hbm.at[pl.ds(base + j * _G, _G)])


@functools.cache
def _gather_rows_fn():
    @functools.partial(
        pl.kernel,
        out_type=jax.ShapeDtypeStruct((_BL, D_MODEL), jnp.float32),
        mesh=plsc.VectorSubcoreMesh(core_axis_name="c", subcore_axis_name="s"),
        scratch_types=[
            pltpu.VMEM((_CH, _G), jnp.int32),
            pltpu.VMEM((_G, D_MODEL), jnp.float32),
            pltpu.VMEM((_G, D_MODEL), jnp.float32),
            pltpu.VMEM((_G, D_MODEL), jnp.float32),
            pltpu.VMEM((_G, D_MODEL), jnp.float32),
            pltpu.SemaphoreType.DMA,
            pltpu.SemaphoreType.DMA,
            pltpu.SemaphoreType.DMA,
            pltpu.SemaphoreType.DMA,
        ],
    )
    def _gather_rows(ids_hbm, table_hbm, out_hbm, *scratch):
        _gather_body(ids_hbm, table_hbm, out_hbm, *scratch)

    return _gather_rows


def kernel(dense_ids, padded_table, W, b):
    proj = _project_table(padded_table, W, b)
    ids = (dense_ids.astype(jnp.int32) + 1).reshape(_NW, _CH, _G)
    rows = _gather_rows_fn()(ids, proj)
    return rows.reshape(BATCH, HIST, D_MODEL)


# speedup vs baseline: 9.4305x; 1.0419x over previous
import functools

import jax
import jax.numpy as jnp
from jax import lax
from jax.experimental import pallas as pl
from jax.experimental.pallas import tpu as pltpu
from jax.experimental.pallas import tpu_sc as plsc

N_ITEMS = 100000
AUDIO_DIM = 128
D_MODEL = 128
BATCH = 4096
HIST = 200


_BLK = 4096


def _project_body(x_ref, w_ref, b_ref, o_ref):
    x = x_ref[...]
    w = w_ref[...]
    y = lax.dot_general(x, w, (((1,), (1,)), ((), ())),
                        preferred_element_type=jnp.float32)
    y = y + b_ref[...]
    ss = jnp.sum(y * y, axis=1, keepdims=True)
    o_ref[...] = y * lax.rsqrt(jnp.maximum(ss, 1e-24))


def _project_table(padded_table, W, b):
    n_rows = padded_table.shape[0]
    grid = pl.cdiv(n_rows, _BLK)
    return pl.pallas_call(
        _project_body,
        grid=(grid,),
        in_specs=[
            pl.BlockSpec((_BLK, AUDIO_DIM), lambda i: (i, 0)),
            pl.BlockSpec((D_MODEL, AUDIO_DIM), lambda i: (0, 0)),
            pl.BlockSpec((1, D_MODEL), lambda i: (0, 0)),
        ],
        out_specs=pl.BlockSpec((_BLK, D_MODEL), lambda i: (i, 0)),
        out_shape=jax.ShapeDtypeStruct((n_rows, D_MODEL), jnp.float32),
    )(padded_table, W, b.reshape(1, D_MODEL))


_NC, _NS = 2, 16
_NW = _NC * _NS
_BL = BATCH * HIST
_G = 128
_CH = _BL // (_NW * _G)
_NBUF = 4


def _gather_body(ids_hbm, table_hbm, out_hbm, idx_v,
                 b0, b1, b2, b3, s0, s1, s2, s3):
    bufs = (b0, b1, b2, b3)
    sems = (s0, s1, s2, s3)
    wid = lax.axis_index("s") * _NC + lax.axis_index("c")
    base = wid * (_CH * _G)

    pltpu.sync_copy(ids_hbm.at[wid], idx_v)

    for b in range(_NBUF):
        pltpu.async_copy(table_hbm.at[idx_v.at[b]], bufs[b], sems[b])

    def outer(i, carry):
        j0 = i * _NBUF
        for b in range(_NBUF):
            j = j0 + b
            pltpu.make_async_copy(table_hbm.at[idx_v.at[j]],
                                  bufs[b], sems[b]).wait()
            pltpu.sync_copy(bufs[b], out_hbm.at[pl.ds(base + j * _G, _G)])
            pltpu.async_copy(table_hbm.at[idx_v.at[j + _NBUF]],
                             bufs[b], sems[b])
        return carry

    lax.fori_loop(0, (_CH - _NBUF) // _NBUF, outer, 0)

    j0 = _CH - _NBUF
    for b in range(_NBUF):
        j = j0 + b
        pltpu.make_async_copy(table_hbm.at[idx_v.at[j]],
                              bufs[b], sems[b]).wait()
        pltpu.sync_copy(bufs[b], out_hbm.at[pl.ds(base + j * _G, _G)])


@functools.cache
def _gather_rows_fn():
    @functools.partial(
        pl.kernel,
        out_type=jax.ShapeDtypeStruct((_BL, D_MODEL), jnp.float32),
        mesh=plsc.VectorSubcoreMesh(core_axis_name="c", subcore_axis_name="s"),
        scratch_types=[
            pltpu.VMEM((_CH, _G), jnp.int32),
            pltpu.VMEM((_G, D_MODEL), jnp.float32),
            pltpu.VMEM((_G, D_MODEL), jnp.float32),
            pltpu.VMEM((_G, D_MODEL), jnp.float32),
            pltpu.VMEM((_G, D_MODEL), jnp.float32),
            pltpu.SemaphoreType.DMA,
            pltpu.SemaphoreType.DMA,
            pltpu.SemaphoreType.DMA,
            pltpu.SemaphoreType.DMA,
        ],
    )
    def _gather_rows(ids_hbm, table_hbm, out_hbm, *scratch):
        _gather_body(ids_hbm, table_hbm, out_hbm, *scratch)

    return _gather_rows


def kernel(dense_ids, padded_table, W, b):
    proj = _project_table(padded_table, W, b)
    ids = (dense_ids.astype(jnp.int32) + 1).reshape(_NW, _CH, _G)
    rows = _gather_rows_fn()(ids, proj)
    return rows.reshape(BATCH, HIST, D_MODEL)


# speedup vs baseline: 9.4681x; 1.0040x over previous
import functools

import jax
import jax.numpy as jnp
from jax import lax
from jax.experimental import pallas as pl
from jax.experimental.pallas import tpu as pltpu
from jax.experimental.pallas import tpu_sc as plsc

N_ITEMS = 100000
AUDIO_DIM = 128
D_MODEL = 128
BATCH = 4096
HIST = 200


_BLK = 4096


def _project_body(x_ref, w_ref, b_ref, o_ref):
    x = x_ref[...]
    w = w_ref[...]
    y = lax.dot_general(x, w, (((1,), (1,)), ((), ())),
                        preferred_element_type=jnp.float32)
    y = y + b_ref[...]
    ss = jnp.sum(y * y, axis=1, keepdims=True)
    o_ref[...] = y * lax.rsqrt(jnp.maximum(ss, 1e-24))


def _project_table(padded_table, W, b):
    n_rows = padded_table.shape[0]
    grid = pl.cdiv(n_rows, _BLK)
    return pl.pallas_call(
        _project_body,
        grid=(grid,),
        in_specs=[
            pl.BlockSpec((_BLK, AUDIO_DIM), lambda i: (i, 0)),
            pl.BlockSpec((D_MODEL, AUDIO_DIM), lambda i: (0, 0)),
            pl.BlockSpec((1, D_MODEL), lambda i: (0, 0)),
        ],
        out_specs=pl.BlockSpec((_BLK, D_MODEL), lambda i: (i, 0)),
        out_shape=jax.ShapeDtypeStruct((n_rows, D_MODEL), jnp.float32),
    )(padded_table, W, b.reshape(1, D_MODEL))


_NC, _NS = 2, 16
_NW = _NC * _NS
_BL = BATCH * HIST
_G = 128
_CH = _BL // (_NW * _G)
_NBUF = 6
_LA = 4


def _gather_body(ids_hbm, table_hbm, out_hbm, idx_v, bufs, gs, ws):
    wid = lax.axis_index("s") * _NC + lax.axis_index("c")
    base = wid * (_CH * _G)

    pltpu.sync_copy(ids_hbm.at[wid], idx_v)

    def gstart(k, b):
        pltpu.async_copy(table_hbm.at[idx_v.at[k]], bufs[b], gs[b])

    def gwait(k, b):
        pltpu.make_async_copy(table_hbm.at[idx_v.at[k]], bufs[b], gs[b]).wait()

    def wstart(j, b):
        pltpu.async_copy(bufs[b], out_hbm.at[pl.ds(base + j * _G, _G)], ws[b])

    def wwait(j, b):
        pltpu.make_async_copy(bufs[b],
                              out_hbm.at[pl.ds(base + j * _G, _G)],
                              ws[b]).wait()

    for k in range(_LA):
        gstart(k, k % _NBUF)

    def step(j, b):
        k = j + _LA
        wwait(k - _NBUF, (b + _LA) % _NBUF)
        gstart(k, (b + _LA) % _NBUF)
        gwait(j, b)
        wstart(j, b)

    for j in range(_NBUF):
        k = j + _LA
        if k >= _NBUF:
            wwait(k - _NBUF, k % _NBUF)
        gstart(k, k % _NBUF)
        gwait(j, j % _NBUF)
        wstart(j, j % _NBUF)

    steady_end = _CH - ((_CH - _NBUF) % _NBUF) - _NBUF
    n_blocks = (steady_end - _NBUF) // _NBUF

    def outer(i, carry):
        j0 = _NBUF + i * _NBUF
        for b in range(_NBUF):
            step(j0 + b, b)
        return carry

    lax.fori_loop(0, n_blocks, outer, 0)

    for j in range(steady_end, _CH):
        k = j + _LA
        if k < _CH:
            wwait(k - _NBUF, k % _NBUF)
            gstart(k, k % _NBUF)
        gwait(j, j % _NBUF)
        wstart(j, j % _NBUF)

    for j in range(_CH - _NBUF, _CH):
        wwait(j, j % _NBUF)


@functools.cache
def _gather_rows_fn():
    @functools.partial(
        pl.kernel,
        out_type=jax.ShapeDtypeStruct((_BL, D_MODEL), jnp.float32),
        mesh=plsc.VectorSubcoreMesh(core_axis_name="c", subcore_axis_name="s"),
        scratch_types=(
            [pltpu.VMEM((_CH, _G), jnp.int32)]
            + [pltpu.VMEM((_G, D_MODEL), jnp.float32)] * _NBUF
            + [pltpu.SemaphoreType.DMA] * (2 * _NBUF)
        ),
    )
    def _gather_rows(ids_hbm, table_hbm, out_hbm, idx_v, *scratch):
        bufs = scratch[:_NBUF]
        gs = scratch[_NBUF:2 * _NBUF]
        ws = scratch[2 * _NBUF:]
        _gather_body(ids_hbm, table_hbm, out_hbm, idx_v, bufs, gs, ws)

    return _gather_rows


def kernel(dense_ids, padded_table, W, b):
    proj = _project_table(padded_table, W, b)
    ids = (dense_ids.astype(jnp.int32) + 1).reshape(_NW, _CH, _G)
    rows = _gather_rows_fn()(ids, proj)
    return rows.reshape(BATCH, HIST, D_MODEL)


# speedup vs baseline: 9.4740x; 1.0006x over previous
import functools

import jax
import jax.numpy as jnp
from jax import lax
from jax.experimental import pallas as pl
from jax.experimental.pallas import tpu as pltpu
from jax.experimental.pallas import tpu_sc as plsc

N_ITEMS = 100000
AUDIO_DIM = 128
D_MODEL = 128
BATCH = 4096
HIST = 200


_BLK = 4096


def _project_body(x_ref, w_ref, b_ref, o_ref):
    x = x_ref[...]
    w = w_ref[...]
    y = lax.dot_general(x, w, (((1,), (1,)), ((), ())),
                        preferred_element_type=jnp.float32)
    y = y + b_ref[...]
    ss = jnp.sum(y * y, axis=1, keepdims=True)
    o_ref[...] = y * lax.rsqrt(jnp.maximum(ss, 1e-24))


def _project_table(padded_table, W, b):
    n_rows = padded_table.shape[0]
    grid = pl.cdiv(n_rows, _BLK)
    return pl.pallas_call(
        _project_body,
        grid=(grid,),
        in_specs=[
            pl.BlockSpec((_BLK, AUDIO_DIM), lambda i: (i, 0)),
            pl.BlockSpec((D_MODEL, AUDIO_DIM), lambda i: (0, 0)),
            pl.BlockSpec((1, D_MODEL), lambda i: (0, 0)),
        ],
        out_specs=pl.BlockSpec((_BLK, D_MODEL), lambda i: (i, 0)),
        out_shape=jax.ShapeDtypeStruct((n_rows, D_MODEL), jnp.float32),
    )(padded_table, W, b.reshape(1, D_MODEL))


_NC, _NS = 2, 16
_NW = _NC * _NS
_BL = BATCH * HIST
_G = 128
_CH = _BL // (_NW * _G)
_NBUF = 6
_LA = 5


def _gather_body(ids_hbm, table_hbm, out_hbm, idx_v, bufs, gs, ws):
    wid = lax.axis_index("s") * _NC + lax.axis_index("c")
    base = wid * (_CH * _G)

    pltpu.sync_copy(ids_hbm.at[wid], idx_v)

    def gstart(k, b):
        pltpu.async_copy(table_hbm.at[idx_v.at[k]], bufs[b], gs[b])

    def gwait(k, b):
        pltpu.make_async_copy(table_hbm.at[idx_v.at[k]], bufs[b], gs[b]).wait()

    def wstart(j, b):
        pltpu.async_copy(bufs[b], out_hbm.at[pl.ds(base + j * _G, _G)], ws[b])

    def wwait(j, b):
        pltpu.make_async_copy(bufs[b],
                              out_hbm.at[pl.ds(base + j * _G, _G)],
                              ws[b]).wait()

    for k in range(_LA):
        gstart(k, k % _NBUF)

    def step(j, b):
        k = j + _LA
        wwait(k - _NBUF, (b + _LA) % _NBUF)
        gstart(k, (b + _LA) % _NBUF)
        gwait(j, b)
        wstart(j, b)

    for j in range(_NBUF):
        k = j + _LA
        if k >= _NBUF:
            wwait(k - _NBUF, k % _NBUF)
        gstart(k, k % _NBUF)
        gwait(j, j % _NBUF)
        wstart(j, j % _NBUF)

    steady_end = _CH - ((_CH - _NBUF) % _NBUF) - _NBUF
    n_blocks = (steady_end - _NBUF) // _NBUF

    def outer(i, carry):
        j0 = _NBUF + i * _NBUF
        for b in range(_NBUF):
            step(j0 + b, b)
        return carry

    lax.fori_loop(0, n_blocks, outer, 0)

    for j in range(steady_end, _CH):
        k = j + _LA
        if k < _CH:
            wwait(k - _NBUF, k % _NBUF)
            gstart(k, k % _NBUF)
        gwait(j, j % _NBUF)
        wstart(j, j % _NBUF)

    for j in range(_CH - _NBUF, _CH):
        wwait(j, j % _NBUF)


@functools.cache
def _gather_rows_fn():
    @functools.partial(
        pl.kernel,
        out_type=jax.ShapeDtypeStruct((_BL, D_MODEL), jnp.float32),
        mesh=plsc.VectorSubcoreMesh(core_axis_name="c", subcore_axis_name="s"),
        scratch_types=(
            [pltpu.VMEM((_CH, _G), jnp.int32)]
            + [pltpu.VMEM((_G, D_MODEL), jnp.float32)] * _NBUF
            + [pltpu.SemaphoreType.DMA] * (2 * _NBUF)
        ),
    )
    def _gather_rows(ids_hbm, table_hbm, out_hbm, idx_v, *scratch):
        bufs = scratch[:_NBUF]
        gs = scratch[_NBUF:2 * _NBUF]
        ws = scratch[2 * _NBUF:]
        _gather_body(ids_hbm, table_hbm, out_hbm, idx_v, bufs, gs, ws)

    return _gather_rows


def kernel(dense_ids, padded_table, W, b):
    proj = _project_table(padded_table, W, b)
    ids = (dense_ids.astype(jnp.int32) + 1).reshape(_NW, _CH, _G)
    rows = _gather_rows_fn()(ids, proj)
    return rows.reshape(BATCH, HIST, D_MODEL)


# speedup vs baseline: 9.5613x; 1.0092x over previous
import functools

import jax
import jax.numpy as jnp
from jax import lax
from jax.experimental import pallas as pl
from jax.experimental.pallas import tpu as pltpu
from jax.experimental.pallas import tpu_sc as plsc

N_ITEMS = 100000
AUDIO_DIM = 128
D_MODEL = 128
BATCH = 4096
HIST = 200


_BLK = 4096


def _project_body(x_ref, w_ref, b_ref, o_ref):
    x = x_ref[...]
    w = w_ref[...]
    y = lax.dot_general(x, w, (((1,), (1,)), ((), ())),
                        preferred_element_type=jnp.float32)
    y = y + b_ref[...]
    ss = jnp.sum(y * y, axis=1, keepdims=True)
    o_ref[...] = y * lax.rsqrt(jnp.maximum(ss, 1e-24))


def _project_table(padded_table, W, b):
    n_rows = padded_table.shape[0]
    grid = pl.cdiv(n_rows, _BLK)
    return pl.pallas_call(
        _project_body,
        grid=(grid,),
        in_specs=[
            pl.BlockSpec((_BLK, AUDIO_DIM), lambda i: (i, 0)),
            pl.BlockSpec((D_MODEL, AUDIO_DIM), lambda i: (0, 0)),
            pl.BlockSpec((1, D_MODEL), lambda i: (0, 0)),
        ],
        out_specs=pl.BlockSpec((_BLK, D_MODEL), lambda i: (i, 0)),
        out_shape=jax.ShapeDtypeStruct((n_rows, D_MODEL), jnp.float32),
    )(padded_table, W, b.reshape(1, D_MODEL))


_NC, _NS = 2, 16
_NW = _NC * _NS
_BL = BATCH * HIST
_G = 128
_CH = _BL // (_NW * _G)
_NBUF = 6
_LA = 4


def _gather_body(ids_hbm, table_hbm, out_hbm, idx_v, bufs, gs, ws):
    wid = lax.axis_index("s") * _NC + lax.axis_index("c")
    base = wid * (_CH * _G)

    pltpu.sync_copy(ids_hbm.at[wid], idx_v)

    def gstart(k, b):
        for h in range(_G // 16):
            sl = pl.ds(h * 16, 16)
            idx_v[k, sl] = idx_v[k, sl] + 1
        pltpu.async_copy(table_hbm.at[idx_v.at[k]], bufs[b], gs[b])

    def gwait(k, b):
        pltpu.make_async_copy(table_hbm.at[idx_v.at[k]], bufs[b], gs[b]).wait()

    def wstart(j, b):
        pltpu.async_copy(bufs[b], out_hbm.at[pl.ds(base + j * _G, _G)], ws[b])

    def wwait(j, b):
        pltpu.make_async_copy(bufs[b],
                              out_hbm.at[pl.ds(base + j * _G, _G)],
                              ws[b]).wait()

    for k in range(_LA):
        gstart(k, k % _NBUF)

    def step(j, b):
        k = j + _LA
        wwait(k - _NBUF, (b + _LA) % _NBUF)
        gstart(k, (b + _LA) % _NBUF)
        gwait(j, b)
        wstart(j, b)

    for j in range(_NBUF):
        k = j + _LA
        if k >= _NBUF:
            wwait(k - _NBUF, k % _NBUF)
        gstart(k, k % _NBUF)
        gwait(j, j % _NBUF)
        wstart(j, j % _NBUF)

    steady_end = _CH - ((_CH - _NBUF) % _NBUF) - _NBUF
    n_blocks = (steady_end - _NBUF) // _NBUF

    def outer(i, carry):
        j0 = _NBUF + i * _NBUF
        for b in range(_NBUF):
            step(j0 + b, b)
        return carry

    lax.fori_loop(0, n_blocks, outer, 0)

    for j in range(steady_end, _CH):
        k = j + _LA
        if k < _CH:
            wwait(k - _NBUF, k % _NBUF)
            gstart(k, k % _NBUF)
        gwait(j, j % _NBUF)
        wstart(j, j % _NBUF)

    for j in range(_CH - _NBUF, _CH):
        wwait(j, j % _NBUF)


@functools.cache
def _gather_rows_fn():
    @functools.partial(
        pl.kernel,
        out_type=jax.ShapeDtypeStruct((_BL, D_MODEL), jnp.float32),
        mesh=plsc.VectorSubcoreMesh(core_axis_name="c", subcore_axis_name="s"),
        scratch_types=(
            [pltpu.VMEM((_CH, _G), jnp.int32)]
            + [pltpu.VMEM((_G, D_MODEL), jnp.float32)] * _NBUF
            + [pltpu.SemaphoreType.DMA] * (2 * _NBUF)
        ),
    )
    def _gather_rows(ids_hbm, table_hbm, out_hbm, idx_v, *scratch):
        bufs = scratch[:_NBUF]
        gs = scratch[_NBUF:2 * _NBUF]
        ws = scratch[2 * _NBUF:]
        _gather_body(ids_hbm, table_hbm, out_hbm, idx_v, bufs, gs, ws)

    return _gather_rows


def kernel(dense_ids, padded_table, W, b):
    proj = _project_table(padded_table, W, b)
    ids = dense_ids.astype(jnp.int32).reshape(_NW, _CH, _G)
    rows = _gather_rows_fn()(ids, proj)
    return rows.reshape(BATCH, HIST, D_MODEL)


# speedup vs baseline: 9.7692x; 1.0217x over previous
import functools

import jax
import jax.numpy as jnp
from jax import lax
from jax.experimental import pallas as pl
from jax.experimental.pallas import tpu as pltpu
from jax.experimental.pallas import tpu_sc as plsc

N_ITEMS = 100000
AUDIO_DIM = 128
D_MODEL = 128
BATCH = 4096
HIST = 200


_BLK = 8192


def _project_body(x_ref, w_ref, b_ref, o_ref):
    x = x_ref[...]
    w = w_ref[...]
    y = lax.dot_general(x, w, (((1,), (1,)), ((), ())),
                        preferred_element_type=jnp.float32)
    y = y + b_ref[...]
    ss = jnp.sum(y * y, axis=1, keepdims=True)
    o_ref[...] = y * lax.rsqrt(jnp.maximum(ss, 1e-24))


def _project_table(padded_table, W, b):
    n_rows = padded_table.shape[0]
    grid = pl.cdiv(n_rows, _BLK)
    return pl.pallas_call(
        _project_body,
        grid=(grid,),
        in_specs=[
            pl.BlockSpec((_BLK, AUDIO_DIM), lambda i: (i, 0)),
            pl.BlockSpec((D_MODEL, AUDIO_DIM), lambda i: (0, 0)),
            pl.BlockSpec((1, D_MODEL), lambda i: (0, 0)),
        ],
        out_specs=pl.BlockSpec((_BLK, D_MODEL), lambda i: (i, 0)),
        out_shape=jax.ShapeDtypeStruct((n_rows, D_MODEL), jnp.float32),
    )(padded_table, W, b.reshape(1, D_MODEL))


_NC, _NS = 2, 16
_NW = _NC * _NS
_BL = BATCH * HIST
_G = 128
_CH = _BL // (_NW * _G)
_NBUF = 6
_LA = 4


def _gather_body(ids_hbm, table_hbm, out_hbm, idx_v, bufs, gs, ws):
    wid = lax.axis_index("s") * _NC + lax.axis_index("c")
    base = wid * (_CH * _G)

    pltpu.sync_copy(ids_hbm.at[wid], idx_v)

    def gstart(k, b):
        for h in range(_G // 16):
            sl = pl.ds(h * 16, 16)
            idx_v[k, sl] = idx_v[k, sl] + 1
        pltpu.async_copy(table_hbm.at[idx_v.at[k]], bufs[b], gs[b])

    def gwait(k, b):
        pltpu.make_async_copy(table_hbm.at[idx_v.at[k]], bufs[b], gs[b]).wait()

    def wstart(j, b):
        pltpu.async_copy(bufs[b], out_hbm.at[pl.ds(base + j * _G, _G)], ws[b])

    def wwait(j, b):
        pltpu.make_async_copy(bufs[b],
                              out_hbm.at[pl.ds(base + j * _G, _G)],
                              ws[b]).wait()

    for k in range(_LA):
        gstart(k, k % _NBUF)

    def step(j, b):
        k = j + _LA
        wwait(k - _NBUF, (b + _LA) % _NBUF)
        gstart(k, (b + _LA) % _NBUF)
        gwait(j, b)
        wstart(j, b)

    for j in range(_NBUF):
        k = j + _LA
        if k >= _NBUF:
            wwait(k - _NBUF, k % _NBUF)
        gstart(k, k % _NBUF)
        gwait(j, j % _NBUF)
        wstart(j, j % _NBUF)

    steady_end = _CH - ((_CH - _NBUF) % _NBUF) - _NBUF
    n_blocks = (steady_end - _NBUF) // _NBUF

    def outer(i, carry):
        j0 = _NBUF + i * _NBUF
        for b in range(_NBUF):
            step(j0 + b, b)
        return carry

    lax.fori_loop(0, n_blocks, outer, 0)

    for j in range(steady_end, _CH):
        k = j + _LA
        if k < _CH:
            wwait(k - _NBUF, k % _NBUF)
            gstart(k, k % _NBUF)
        gwait(j, j % _NBUF)
        wstart(j, j % _NBUF)

    for j in range(_CH - _NBUF, _CH):
        wwait(j, j % _NBUF)


@functools.cache
def _gather_rows_fn():
    @functools.partial(
        pl.kernel,
        out_type=jax.ShapeDtypeStruct((_BL, D_MODEL), jnp.float32),
        mesh=plsc.VectorSubcoreMesh(core_axis_name="c", subcore_axis_name="s"),
        scratch_types=(
            [pltpu.VMEM((_CH, _G), jnp.int32)]
            + [pltpu.VMEM((_G, D_MODEL), jnp.float32)] * _NBUF
            + [pltpu.SemaphoreType.DMA] * (2 * _NBUF)
        ),
    )
    def _gather_rows(ids_hbm, table_hbm, out_hbm, idx_v, *scratch):
        bufs = scratch[:_NBUF]
        gs = scratch[_NBUF:2 * _NBUF]
        ws = scratch[2 * _NBUF:]
        _gather_body(ids_hbm, table_hbm, out_hbm, idx_v, bufs, gs, ws)

    return _gather_rows


def kernel(dense_ids, padded_table, W, b):
    proj = _project_table(padded_table, W, b)
    ids = dense_ids.astype(jnp.int32).reshape(_NW, _CH, _G)
    rows = _gather_rows_fn()(ids, proj)
    return rows.reshape(BATCH, HIST, D_MODEL)


# speedup vs baseline: 9.8077x; 1.0039x over previous
import functools

import jax
import jax.numpy as jnp
from jax import lax
from jax.experimental import pallas as pl
from jax.experimental.pallas import tpu as pltpu
from jax.experimental.pallas import tpu_sc as plsc

N_ITEMS = 100000
AUDIO_DIM = 128
D_MODEL = 128
BATCH = 4096
HIST = 200


_BLK = 16384


def _project_body(x_ref, w_ref, b_ref, o_ref):
    x = x_ref[...]
    w = w_ref[...]
    y = lax.dot_general(x, w, (((1,), (1,)), ((), ())),
                        preferred_element_type=jnp.float32)
    y = y + b_ref[...]
    ss = jnp.sum(y * y, axis=1, keepdims=True)
    o_ref[...] = y * lax.rsqrt(jnp.maximum(ss, 1e-24))


def _project_table(padded_table, W, b):
    n_rows = padded_table.shape[0]
    grid = pl.cdiv(n_rows, _BLK)
    return pl.pallas_call(
        _project_body,
        grid=(grid,),
        in_specs=[
            pl.BlockSpec((_BLK, AUDIO_DIM), lambda i: (i, 0)),
            pl.BlockSpec((D_MODEL, AUDIO_DIM), lambda i: (0, 0)),
            pl.BlockSpec((1, D_MODEL), lambda i: (0, 0)),
        ],
        out_specs=pl.BlockSpec((_BLK, D_MODEL), lambda i: (i, 0)),
        out_shape=jax.ShapeDtypeStruct((n_rows, D_MODEL), jnp.float32),
    )(padded_table, W, b.reshape(1, D_MODEL))


_NC, _NS = 2, 16
_NW = _NC * _NS
_BL = BATCH * HIST
_G = 128
_CH = _BL // (_NW * _G)
_NBUF = 6
_LA = 4


def _gather_body(ids_hbm, table_hbm, out_hbm, idx_v, bufs, gs, ws):
    wid = lax.axis_index("s") * _NC + lax.axis_index("c")
    base = wid * (_CH * _G)

    pltpu.sync_copy(ids_hbm.at[wid], idx_v)

    def gstart(k, b):
        for h in range(_G // 16):
            sl = pl.ds(h * 16, 16)
            idx_v[k, sl] = idx_v[k, sl] + 1
        pltpu.async_copy(table_hbm.at[idx_v.at[k]], bufs[b], gs[b])

    def gwait(k, b):
        pltpu.make_async_copy(table_hbm.at[idx_v.at[k]], bufs[b], gs[b]).wait()

    def wstart(j, b):
        pltpu.async_copy(bufs[b], out_hbm.at[pl.ds(base + j * _G, _G)], ws[b])

    def wwait(j, b):
        pltpu.make_async_copy(bufs[b],
                              out_hbm.at[pl.ds(base + j * _G, _G)],
                              ws[b]).wait()

    for k in range(_LA):
        gstart(k, k % _NBUF)

    def step(j, b):
        k = j + _LA
        wwait(k - _NBUF, (b + _LA) % _NBUF)
        gstart(k, (b + _LA) % _NBUF)
        gwait(j, b)
        wstart(j, b)

    for j in range(_NBUF):
        k = j + _LA
        if k >= _NBUF:
            wwait(k - _NBUF, k % _NBUF)
        gstart(k, k % _NBUF)
        gwait(j, j % _NBUF)
        wstart(j, j % _NBUF)

    steady_end = _CH - ((_CH - _NBUF) % _NBUF) - _NBUF
    n_blocks = (steady_end - _NBUF) // _NBUF

    def outer(i, carry):
        j0 = _NBUF + i * _NBUF
        for b in range(_NBUF):
            step(j0 + b, b)
        return carry

    lax.fori_loop(0, n_blocks, outer, 0)

    for j in range(steady_end, _CH):
        k = j + _LA
        if k < _CH:
            wwait(k - _NBUF, k % _NBUF)
            gstart(k, k % _NBUF)
        gwait(j, j % _NBUF)
        wstart(j, j % _NBUF)

    for j in range(_CH - _NBUF, _CH):
        wwait(j, j % _NBUF)


@functools.cache
def _gather_rows_fn():
    @functools.partial(
        pl.kernel,
        out_type=jax.ShapeDtypeStruct((_BL, D_MODEL), jnp.float32),
        mesh=plsc.VectorSubcoreMesh(core_axis_name="c", subcore_axis_name="s"),
        scratch_types=(
            [pltpu.VMEM((_CH, _G), jnp.int32)]
            + [pltpu.VMEM((_G, D_MODEL), jnp.float32)] * _NBUF
            + [pltpu.SemaphoreType.DMA] * (2 * _NBUF)
        ),
    )
    def _gather_rows(ids_hbm, table_hbm, out_hbm, idx_v, *scratch):
        bufs = scratch[:_NBUF]
        gs = scratch[_NBUF:2 * _NBUF]
        ws = scratch[2 * _NBUF:]
        _gather_body(ids_hbm, table_hbm, out_hbm, idx_v, bufs, gs, ws)

    return _gather_rows


def kernel(dense_ids, padded_table, W, b):
    proj = _project_table(padded_table, W, b)
    ids = dense_ids.astype(jnp.int32).reshape(_NW, _CH, _G)
    rows = _gather_rows_fn()(ids, proj)
    return rows.reshape(BATCH, HIST, D_MODEL)
